# baseline (speedup 1.0000x reference)
import jax
import jax.numpy as jnp
from jax import lax
from jax.experimental import pallas as pl
from jax.experimental.pallas import tpu as pltpu
from jax.experimental.pallas import tpu_sc as plsc

D = 1024
B = 4
S = 4096
NC = 2
NS = 16
NW = NC * NS
SEQ_PER_W = S // NW
CHUNK = 16
NCHUNK = SEQ_PER_W // CHUNK
NTILE = NCHUNK * B
NVREG = CHUNK * D // 16


def _embed_body(idx_hbm, tok_hbm, pos_hbm, out_hbm,
                idx_v, tok0, tok1, tok2, tok3, posA, posB,
                gs0, gs1, gs2, gs3, ss0, ss1, ss2, ss3, psA, psB):
    wid = lax.axis_index("s") * NC + lax.axis_index("c")
    s0 = wid * SEQ_PER_W
    tok_b = [tok0, tok1, tok2, tok3]
    pos_b = [posA, posB]
    gsem = [gs0, gs1, gs2, gs3]
    ssem = [ss0, ss1, ss2, ss3]
    psem = [psA, psB]

    for b in range(B):
        pltpu.sync_copy(idx_hbm.at[pl.ds(b * S + s0, SEQ_PER_W)],
                        idx_v.at[b])

    def start_gather(g, b):
        pltpu.async_copy(
            tok_hbm.at[idx_v.at[b, pl.ds(g * CHUNK, CHUNK)]],
            tok_b[b], gsem[b])

    def wait_gather(b):
        pltpu.make_async_copy(
            pos_hbm.at[pl.ds(0, CHUNK)], tok_b[b], gsem[b]).wait()

    def start_pos(g, pb):
        pltpu.async_copy(
            pos_hbm.at[pl.ds(s0 + g * CHUNK, CHUNK)], pos_b[pb], psem[pb])

    def wait_pos(pb):
        pltpu.make_async_copy(
            pos_hbm.at[pl.ds(0, CHUNK)], pos_b[pb], psem[pb]).wait()

    def start_store(g, b):
        pltpu.async_copy(
            tok_b[b], out_hbm.at[pl.ds(b * S + s0 + g * CHUNK, CHUNK)],
            ssem[b])

    def wait_store(b):
        pltpu.make_async_copy(
            tok_b[b], out_hbm.at[pl.ds(0, CHUNK)], ssem[b]).wait()

    def compute(b, pb):
        tv, pv = tok_b[b], pos_b[pb]

        @plsc.parallel_loop(0, NVREG, unroll=8)
        def _(t):
            r = lax.shift_right_logical(t, 6)
            col = lax.shift_left(lax.bitwise_and(t, 63), 4)
            sl = pl.ds(pl.multiple_of(col, 16), 16)
            plsc.addupdate(tv.at[r, sl], pv[r, sl])

    def tile(g, b, pb, prep, store_started, pos_prefetch):
        wait_gather(b)
        if b == 0:
            wait_pos(pb)
        compute(b, pb)
        start_store(g, b)
        if b == B - 1 and pos_prefetch:
            start_pos(g + 2, pb)
        if prep:
            b2 = (b + 2) % B
            gn = g + (b + 2) // B
            if store_started:
                wait_store(b2)
            start_gather(gn, b2)

    start_pos(0, 0)
    start_pos(1, 1)
    start_gather(0, 0)
    start_gather(0, 1)

    for b in range(B):
        tile(0, b, 0, prep=True, store_started=(b >= 2), pos_prefetch=True)

    def pair_body(gg, _):
        g = 1 + gg * 2
        for g2 in range(2):
            pb = (1 + g2) % 2
            for b in range(B):
                tile(g + g2, b, pb, prep=True, store_started=True,
                     pos_prefetch=True)
        return 0

    lax.fori_loop(0, 2, pair_body, 0)

    for g in (5, 6):
        for b in range(B):
            tile(g, b, g % 2, prep=True, store_started=True,
                 pos_prefetch=(g + 2 < NCHUNK))

    for b in range(B):
        tile(NCHUNK - 1, b, (NCHUNK - 1) % 2, prep=(b < 2),
             store_started=True, pos_prefetch=False)

    for b in range(B):
        wait_store(b)


def kernel(idx, token_embd_table, position_embd_table):
    batch, seq = idx.shape
    idx_r = idx.reshape(batch * seq).astype(jnp.int32)
    mesh = plsc.VectorSubcoreMesh(core_axis_name="c", subcore_axis_name="s")
    k = pl.kernel(
        _embed_body,
        mesh=mesh,
        out_type=jax.ShapeDtypeStruct((batch * seq, D), jnp.float32),
        scratch_types=[
            pltpu.VMEM((B, SEQ_PER_W), jnp.int32),
            pltpu.VMEM((CHUNK, D), jnp.float32),
            pltpu.VMEM((CHUNK, D), jnp.float32),
            pltpu.VMEM((CHUNK, D), jnp.float32),
            pltpu.VMEM((CHUNK, D), jnp.float32),
            pltpu.VMEM((CHUNK, D), jnp.float32),
            pltpu.VMEM((CHUNK, D), jnp.float32),
        ] + [pltpu.SemaphoreType.DMA] * 10,
    )
    out = k(idx_r, token_embd_table, position_embd_table)
    return out.reshape(batch, seq, D)


# speedup vs baseline: 1.0452x; 1.0452x over previous
import jax
import jax.numpy as jnp
from jax import lax
from jax.experimental import pallas as pl
from jax.experimental.pallas import tpu as pltpu
from jax.experimental.pallas import tpu_sc as plsc

D = 1024
B = 4
S = 4096
NC = 2
NS = 16
NW = NC * NS
SEQ_PER_W = S // NW
CHUNK = 16
NCHUNK = SEQ_PER_W // CHUNK
NTILE = NCHUNK * B
NVREG = CHUNK * D // 16
NTOK = 5
AHEAD = 3


def _embed_body(idx_hbm, tok_hbm, pos_hbm, out_hbm,
                idx_v, tok0, tok1, tok2, tok3, tok4, posA, posB,
                gs0, gs1, gs2, gs3, gs4, ss0, ss1, ss2, ss3, ss4, psA, psB):
    wid = lax.axis_index("s") * NC + lax.axis_index("c")
    s0 = wid * SEQ_PER_W
    tok_b = [tok0, tok1, tok2, tok3, tok4]
    pos_b = [posA, posB]
    gsem = [gs0, gs1, gs2, gs3, gs4]
    ssem = [ss0, ss1, ss2, ss3, ss4]
    psem = [psA, psB]

    pltpu.sync_copy(idx_hbm.at[wid], idx_v)

    def start_gather(u):
        a = u % NTOK
        g, b = u // B, u % B
        pltpu.async_copy(tok_hbm.at[idx_v.at[u]], tok_b[a], gsem[a])

    def wait_gather(a):
        pltpu.make_async_copy(
            pos_hbm.at[pl.ds(0, CHUNK)], tok_b[a], gsem[a]).wait()

    def start_pos(g):
        pltpu.async_copy(
            pos_hbm.at[pl.ds(s0 + g * CHUNK, CHUNK)], pos_b[g % 2],
            psem[g % 2])

    def wait_pos(pb):
        pltpu.make_async_copy(
            pos_hbm.at[pl.ds(0, CHUNK)], pos_b[pb], psem[pb]).wait()

    def start_store(u):
        a = u % NTOK
        g, b = u // B, u % B
        pltpu.async_copy(
            tok_b[a], out_hbm.at[pl.ds(b * S + s0 + g * CHUNK, CHUNK)],
            ssem[a])

    def wait_store(a):
        pltpu.make_async_copy(
            tok_b[a], out_hbm.at[pl.ds(0, CHUNK)], ssem[a]).wait()

    def compute(a, pb):
        tv, pv = tok_b[a], pos_b[pb]

        @plsc.parallel_loop(0, NVREG, unroll=8)
        def _(t):
            r = lax.shift_right_logical(t, 6)
            col = lax.shift_left(lax.bitwise_and(t, 63), 4)
            sl = pl.ds(pl.multiple_of(col, 16), 16)
            plsc.addupdate(tv.at[r, sl], pv[r, sl])

    start_pos(0)
    start_pos(1)
    for u in range(AHEAD):
        start_gather(u)

    for u in range(NTILE):
        a = u % NTOK
        g, b = u // B, u % B
        wait_gather(a)
        if b == 0:
            wait_pos(g % 2)
        compute(a, g % 2)
        start_store(u)
        if b == B - 1 and g + 2 < NCHUNK:
            start_pos(g + 2)
        un = u + AHEAD
        if un < NTILE:
            an = un % NTOK
            if un >= NTOK:
                wait_store(an)
            start_gather(un)

    for a in range(NTOK):
        wait_store(a)


def kernel(idx, token_embd_table, position_embd_table):
    batch, seq = idx.shape
    idx_r = jnp.transpose(idx.reshape(batch, NW, NCHUNK, CHUNK), (1, 2, 0, 3))
    idx_r = idx_r.reshape(NW, NTILE, CHUNK).astype(jnp.int32)
    mesh = plsc.VectorSubcoreMesh(core_axis_name="c", subcore_axis_name="s")
    k = pl.kernel(
        _embed_body,
        mesh=mesh,
        out_type=jax.ShapeDtypeStruct((batch * seq, D), jnp.float32),
        scratch_types=[
            pltpu.VMEM((NTILE, CHUNK), jnp.int32),
        ] + [pltpu.VMEM((CHUNK, D), jnp.float32)] * (NTOK + 2)
          + [pltpu.SemaphoreType.DMA] * (2 * NTOK + 2),
    )
    out = k(idx_r, token_embd_table, position_embd_table)
    return out.reshape(batch, seq, D)
